# Initial kernel scaffold; baseline (speedup 1.0000x reference)
import jax
import jax.numpy as jnp
from jax.experimental import pallas as pl


def kernel(x, length, W, b, gamma, beta):
    raise NotImplementedError("write your pallas kernel here")


# speedup vs baseline: 3.5561x; 3.5561x over previous
import functools

import jax
import jax.numpy as jnp
from jax import lax
from jax.experimental import pallas as pl
from jax.experimental.pallas import tpu as pltpu
from jax.experimental.pallas import tpu_sc as plsc

TOTAL = 32768
B = 16
C_IN = 32
C_OUT = 128
NW = 32
RPW = 1032
L = 16
PSZ = B * C_IN


def _sc_partial_max(x_hbm, len_hbm, out_hbm, len_vm, buf, part):
    c = lax.axis_index("c")
    s = lax.axis_index("s")
    w = s * 2 + c

    pltpu.sync_copy(len_hbm, len_vm)

    lenv = len_vm[...]
    csum = []
    acc = jnp.int32(0)
    for j in range(B):
        acc = acc + lenv[j]
        csum.append(acc)
    total = csum[-1]

    lo = (w * total) // NW
    hi = ((w + 1) * total) // NW
    base = (jnp.minimum(lo, TOTAL - RPW) // 8) * 8

    pltpu.sync_copy(x_hbm.at[pl.ds(base * C_IN, RPW * C_IN)], buf)

    neg = jnp.full((L,), -jnp.inf, jnp.float32)
    prev = jnp.int32(0)
    for j in range(B):
        r0 = jnp.maximum(prev, lo)
        r1 = jnp.minimum(csum[j], hi)
        prev = csum[j]
        r1 = jnp.maximum(r1, r0)
        p0 = r0 - base
        p1 = r1 - base

        def body(p, carry):
            a0, a1 = carry
            q = p * C_IN
            a0 = jnp.maximum(a0, buf[pl.ds(q, L)])
            a1 = jnp.maximum(a1, buf[pl.ds(q + L, L)])
            return (a0, a1)

        a0, a1 = lax.fori_loop(p0, p1, body, (neg, neg))
        part[pl.ds(j * C_IN, L)] = a0
        part[pl.ds(j * C_IN + L, L)] = a1

    pltpu.sync_copy(part, out_hbm.at[pl.ds(w * PSZ, PSZ)])


@functools.partial(
    pl.kernel,
    out_type=jax.ShapeDtypeStruct((NW * PSZ,), jnp.float32),
    mesh=plsc.VectorSubcoreMesh(core_axis_name="c", subcore_axis_name="s"),
    scratch_types=[
        pltpu.VMEM((B,), jnp.int32),
        pltpu.VMEM((RPW * C_IN,), jnp.float32),
        pltpu.VMEM((PSZ,), jnp.float32),
    ],
)
def _sc_call(x_hbm, len_hbm, out_hbm, len_vm, buf, part):
    _sc_partial_max(x_hbm, len_hbm, out_hbm, len_vm, buf, part)


def _tc_body(part_ref, w_ref, b_ref, g_ref, bt_ref, o_ref):
    pooled = jnp.max(part_ref[...], axis=0)
    y = lax.dot_general(
        pooled, w_ref[...], (((1,), (1,)), ((), ())),
        preferred_element_type=jnp.float32,
    )
    y = y + b_ref[...]
    mean = jnp.mean(y, axis=0, keepdims=True)
    var = jnp.mean(jnp.square(y - mean), axis=0, keepdims=True)
    yn = (y - mean) / jnp.sqrt(var + 1e-5) * g_ref[...] + bt_ref[...]
    o_ref[...] = jnp.maximum(yn, 0.0)


_tc_call = pl.pallas_call(
    _tc_body,
    out_shape=jax.ShapeDtypeStruct((B, C_OUT), jnp.float32),
)


def kernel(x, length, W, b, gamma, beta):
    part = _sc_call(x.reshape(-1), length.astype(jnp.int32))
    part = part.reshape(NW, B, C_IN)
    return _tc_call(part, W, b.reshape(1, C_OUT), gamma.reshape(1, C_OUT),
                    beta.reshape(1, C_OUT))
